# baseline (speedup 1.0000x reference)
import jax
import jax.numpy as jnp
from jax.experimental import pallas as pl

N = 1048576
BLK = 4096


def _softplus(x):
    return jnp.maximum(x, 0.0) + jnp.log1p(jnp.exp(-jnp.abs(x)))


def _sigmoid(x):
    return 1.0 / (1.0 + jnp.exp(-x))


def _volume_kernel(xyz_ref, ynm_ref, we_ref, be_ref, wd_ref, bd_ref,
                   wc_ref, bc_ref, aabb_ref, d_ref, c_ref):
    xyz = xyz_ref[...]
    a0 = aabb_ref[0:1, :]
    a1 = aabb_ref[1:2, :]
    ndc = (xyz - a0) / (a1 - a0) * 2.0 - 1.0
    mask = jnp.all((ndc >= -1.0) & (ndc <= 1.0), axis=-1, keepdims=True)
    f = jnp.maximum(
        jnp.dot(ndc, we_ref[...], preferred_element_type=jnp.float32)
        + be_ref[...], 0.0)
    d = _softplus(
        jnp.dot(f, wd_ref[...], preferred_element_type=jnp.float32)
        + bd_ref[...])
    wc = wc_ref[...]
    c = _sigmoid(
        jnp.dot(f, wc[:16, :], preferred_element_type=jnp.float32)
        + jnp.dot(ynm_ref[...], wc[16:, :], preferred_element_type=jnp.float32)
        + bc_ref[...])
    zero = jnp.float32(0.0)
    d_ref[...] = jnp.where(mask, d, zero)
    c_ref[...] = jnp.where(mask, c, zero)


def kernel(xyz, ynm, W_enc, b_enc, W_d, b_d, W_c, b_c, aabb):
    grid = (N // BLK,)
    out = pl.pallas_call(
        _volume_kernel,
        grid=grid,
        in_specs=[
            pl.BlockSpec((BLK, 3), lambda i: (i, 0)),
            pl.BlockSpec((BLK, 16), lambda i: (i, 0)),
            pl.BlockSpec((3, 16), lambda i: (0, 0)),
            pl.BlockSpec((1, 16), lambda i: (0, 0)),
            pl.BlockSpec((16, 1), lambda i: (0, 0)),
            pl.BlockSpec((1, 1), lambda i: (0, 0)),
            pl.BlockSpec((32, 3), lambda i: (0, 0)),
            pl.BlockSpec((1, 3), lambda i: (0, 0)),
            pl.BlockSpec((2, 3), lambda i: (0, 0)),
        ],
        out_specs=[
            pl.BlockSpec((BLK, 1), lambda i: (i, 0)),
            pl.BlockSpec((BLK, 3), lambda i: (i, 0)),
        ],
        out_shape=[
            jax.ShapeDtypeStruct((N, 1), jnp.float32),
            jax.ShapeDtypeStruct((N, 3), jnp.float32),
        ],
    )(xyz, ynm, W_enc, b_enc.reshape(1, 16), W_d, b_d.reshape(1, 1),
      W_c, b_c.reshape(1, 3), aabb)
    return (out[0], out[1])
